# Initial kernel scaffold; baseline (speedup 1.0000x reference)
import jax
import jax.numpy as jnp
from jax.experimental import pallas as pl


def kernel(input_pos, k_val, v_val, k_cache, v_cache):
    raise NotImplementedError("write your pallas kernel here")


# speedup vs baseline: 26.1816x; 26.1816x over previous
import jax
import jax.numpy as jnp
from jax.experimental import pallas as pl
from jax.experimental.pallas import tpu as pltpu

B = 8
H = 16
S = 2048
D = 128


def _body(pos_ref, kc_ref, vc_ref, kv_ref, vv_ref, ko_ref, vo_ref):
    b = pl.program_id(0)
    h = pl.program_id(1)
    ko_ref[...] = kc_ref[...]
    vo_ref[...] = vc_ref[...]

    @pl.when(h == pos_ref[b] - 1)
    def _():
        ko_ref[0, 0, 0:16, :] = kv_ref[0, 0, :, :]
        vo_ref[0, 0, 0:16, :] = vv_ref[0, 0, :, :]


def kernel(input_pos, k_val, v_val, k_cache, v_cache):
    grid_spec = pltpu.PrefetchScalarGridSpec(
        num_scalar_prefetch=1,
        grid=(B, H),
        in_specs=[
            pl.BlockSpec((1, 1, S, D), lambda b, h, pos: (b, h, 0, 0)),
            pl.BlockSpec((1, 1, S, D), lambda b, h, pos: (b, h, 0, 0)),
            pl.BlockSpec((1, 1, H, D), lambda b, h, pos: (b, 0, 0, 0)),
            pl.BlockSpec((1, 1, H, D), lambda b, h, pos: (b, 0, 0, 0)),
        ],
        out_specs=[
            pl.BlockSpec((1, 1, S, D), lambda b, h, pos: (b, h, 0, 0)),
            pl.BlockSpec((1, 1, S, D), lambda b, h, pos: (b, h, 0, 0)),
        ],
    )
    k_out, v_out = pl.pallas_call(
        _body,
        grid_spec=grid_spec,
        out_shape=[
            jax.ShapeDtypeStruct((B, H, S, D), jnp.float32),
            jax.ShapeDtypeStruct((B, H, S, D), jnp.float32),
        ],
    )(input_pos, k_cache, v_cache, k_val, v_val)
    return (k_out, v_out)


# speedup vs baseline: 28.9219x; 1.1047x over previous
import jax
import jax.numpy as jnp
from jax.experimental import pallas as pl
from jax.experimental.pallas import tpu as pltpu

B = 8
H = 16
S = 2048
D = 128


def _body(pos_ref, kc_ref, vc_ref, kv_ref, vv_ref, ko_ref, vo_ref):
    del kc_ref, vc_ref
    ko_ref[...] = kv_ref[...]
    vo_ref[...] = vv_ref[...]


def kernel(input_pos, k_val, v_val, k_cache, v_cache):
    grid_spec = pltpu.PrefetchScalarGridSpec(
        num_scalar_prefetch=1,
        grid=(B,),
        in_specs=[
            pl.BlockSpec(memory_space=pl.ANY),
            pl.BlockSpec(memory_space=pl.ANY),
            pl.BlockSpec((1, 1, H, D), lambda b, pos: (b, 0, 0, 0)),
            pl.BlockSpec((1, 1, H, D), lambda b, pos: (b, 0, 0, 0)),
        ],
        out_specs=[
            pl.BlockSpec((1, 1, 16, D), lambda b, pos: (b, pos[b] - 1, 0, 0)),
            pl.BlockSpec((1, 1, 16, D), lambda b, pos: (b, pos[b] - 1, 0, 0)),
        ],
    )
    k_out, v_out = pl.pallas_call(
        _body,
        grid_spec=grid_spec,
        out_shape=[
            jax.ShapeDtypeStruct((B, H, S, D), jnp.float32),
            jax.ShapeDtypeStruct((B, H, S, D), jnp.float32),
        ],
        input_output_aliases={1: 0, 2: 1},
    )(input_pos, k_cache, v_cache, k_val, v_val)
    return (k_out, v_out)
